# Initial kernel scaffold; baseline (speedup 1.0000x reference)
import jax
import jax.numpy as jnp
from jax.experimental import pallas as pl


def kernel(test_input, emb_table):
    raise NotImplementedError("write your pallas kernel here")


# speedup vs baseline: 1.1499x; 1.1499x over previous
import functools

import jax
import jax.numpy as jnp
from jax import lax
from jax.experimental import pallas as pl
from jax.experimental.pallas import tpu as pltpu
from jax.experimental.pallas import tpu_sc as plsc

VOCAB = 32128
D_MODEL = 4096

_NC = 2
_NS = 16
_NW = _NC * _NS

_B = 16 * 77
_B_PAD = 1280
_B_PER_W = _B_PAD // _NW
_CHUNK = 8
_NCHUNK = _B_PER_W // _CHUNK


def _emb_body(idx_hbm, table_hbm, out_hbm, idx_v, rows_v, g0, g1, s0, s1):
    wid = lax.axis_index("s") * _NC + lax.axis_index("c")
    base = wid * _B_PER_W
    pltpu.sync_copy(idx_hbm.at[pl.ds(base, _B_PER_W)], idx_v)

    gsems = (g0, g1)
    ssems = (s0, s1)
    gathers = [None, None]
    scatters = [None, None]
    for c in range(_NCHUNK + 1):
        if c < _NCHUNK:
            b = c % 2
            if scatters[b] is not None:
                scatters[b].wait()
            gathers[b] = pltpu.async_copy(
                table_hbm.at[idx_v.at[pl.ds(c * _CHUNK, _CHUNK)]],
                rows_v.at[b],
                gsems[b],
            )
        if c >= 1:
            b = (c - 1) % 2
            gathers[b].wait()
            scatters[b] = pltpu.async_copy(
                rows_v.at[b],
                out_hbm.at[pl.ds(base + (c - 1) * _CHUNK, _CHUNK)],
                ssems[b],
            )
    for b in range(2):
        if scatters[b] is not None:
            scatters[b].wait()


_mesh = plsc.VectorSubcoreMesh(core_axis_name="c", subcore_axis_name="s")

_emb_lookup = functools.partial(
    pl.kernel,
    mesh=_mesh,
    out_type=jax.ShapeDtypeStruct((_B_PAD, D_MODEL), jnp.float32),
    scratch_types=[
        pltpu.VMEM((_B_PER_W,), jnp.int32),
        pltpu.VMEM((2, _CHUNK, D_MODEL), jnp.float32),
        pltpu.SemaphoreType.DMA,
        pltpu.SemaphoreType.DMA,
        pltpu.SemaphoreType.DMA,
        pltpu.SemaphoreType.DMA,
    ],
)(_emb_body)


@jax.jit
def kernel(test_input, emb_table):
    idx = test_input.reshape(-1).astype(jnp.int32)
    idx_pad = jnp.concatenate([idx, jnp.zeros((_B_PAD - _B,), jnp.int32)])
    rows = _emb_lookup(idx_pad, emb_table)
    return rows[:_B].reshape(test_input.shape + (D_MODEL,))


# speedup vs baseline: 1.4362x; 1.2490x over previous
import functools

import jax
import jax.numpy as jnp
from jax import lax
from jax.experimental import pallas as pl
from jax.experimental.pallas import tpu as pltpu
from jax.experimental.pallas import tpu_sc as plsc

VOCAB = 32128
D_MODEL = 4096

_NC = 2
_NS = 16
_NW = _NC * _NS

_B = 16 * 77
_CHUNK = 8
_N_BIG = 26


def _emb_body(idx_hbm, table_hbm, out_hbm, idx_v, rows_v, g0, g1, s0, s1):
    wid = lax.axis_index("s") * _NC + lax.axis_index("c")
    is_big = wid < _N_BIG
    base = jnp.where(is_big, wid * 40, 208 + wid * 32)

    pltpu.sync_copy(idx_hbm.at[pl.ds(base, 32)], idx_v.at[pl.ds(0, 32)])

    @pl.when(is_big)
    def _():
        pltpu.sync_copy(idx_hbm.at[pl.ds(base + 32, 8)], idx_v.at[pl.ds(32, 8)])

    gsems = (g0, g1)
    ssems = (s0, s1)
    gathers = [None, None]
    scatters = [None, None]
    for c in range(5):
        if c < 4:
            b = c % 2
            if scatters[b] is not None:
                scatters[b].wait()
            gathers[b] = pltpu.async_copy(
                table_hbm.at[idx_v.at[pl.ds(c * _CHUNK, _CHUNK)]],
                rows_v.at[b],
                gsems[b],
            )
        if c >= 1:
            b = (c - 1) % 2
            gathers[b].wait()
            scatters[b] = pltpu.async_copy(
                rows_v.at[b],
                out_hbm.at[pl.ds(base + (c - 1) * _CHUNK, _CHUNK)],
                ssems[b],
            )
    scatters[0].wait()

    @pl.when(is_big)
    def _():
        g = pltpu.async_copy(
            table_hbm.at[idx_v.at[pl.ds(4 * _CHUNK, _CHUNK)]],
            rows_v.at[0],
            gsems[0],
        )
        g.wait()
        s = pltpu.async_copy(
            rows_v.at[0],
            out_hbm.at[pl.ds(base + 4 * _CHUNK, _CHUNK)],
            ssems[0],
        )
        s.wait()

    scatters[1].wait()


_mesh = plsc.VectorSubcoreMesh(core_axis_name="c", subcore_axis_name="s")

_emb_lookup = functools.partial(
    pl.kernel,
    mesh=_mesh,
    out_type=jax.ShapeDtypeStruct((_B, D_MODEL), jnp.float32),
    scratch_types=[
        pltpu.VMEM((40,), jnp.int32),
        pltpu.VMEM((2, _CHUNK, D_MODEL), jnp.float32),
        pltpu.SemaphoreType.DMA,
        pltpu.SemaphoreType.DMA,
        pltpu.SemaphoreType.DMA,
        pltpu.SemaphoreType.DMA,
    ],
)(_emb_body)


@jax.jit
def kernel(test_input, emb_table):
    idx = test_input.reshape(-1).astype(jnp.int32)
    rows = _emb_lookup(idx, emb_table)
    return rows.reshape(test_input.shape + (D_MODEL,))


# speedup vs baseline: 1.6225x; 1.1298x over previous
import functools

import jax
import jax.numpy as jnp
import numpy as np
from jax import lax
from jax.experimental import pallas as pl
from jax.experimental.pallas import tpu as pltpu
from jax.experimental.pallas import tpu_sc as plsc

VOCAB = 32128
D_MODEL = 4096

_NC = 2
_NS = 16
_NW = _NC * _NS

_T = 77
_CHUNK = 8
_BLK = 40


def _build_perm():
    perm = np.zeros((_NW * _BLK,), np.int32)
    for w in range(_NW):
        row, part = w // 2, w % 2
        t0 = part * 40
        cnt = 40 if part == 0 else _T - 40
        for j in range(_BLK):
            perm[_BLK * w + j] = _T * row + t0 + min(j, cnt - 1)
    return perm


_PERM = _build_perm()


def _emb_body(idx_hbm, table_hbm, out_hbm, idx_v, rows_v, g0, g1, s0, s1):
    wid = lax.axis_index("s") * _NC + lax.axis_index("c")
    row = wid // 2
    part = wid % 2
    t0 = part * 40

    pltpu.sync_copy(idx_hbm.at[pl.ds(wid * _BLK, _BLK)], idx_v)

    gsems = (g0, g1)
    ssems = (s0, s1)
    gathers = [None, None]
    scatters = [None, None]
    for c in range(5):
        b = c % 2
        if scatters[b] is not None:
            scatters[b].wait()
        gathers[b] = pltpu.async_copy(
            table_hbm.at[idx_v.at[pl.ds(c * _CHUNK, _CHUNK)]],
            rows_v.at[b],
            gsems[b],
        )
        if c >= 1:
            bp = (c - 1) % 2
            gathers[bp].wait()
            scatters[bp] = pltpu.async_copy(
                rows_v.at[bp],
                out_hbm.at[row, pl.ds(t0 + (c - 1) * _CHUNK, _CHUNK)],
                ssems[bp],
            )
    gathers[0].wait()

    @pl.when(part == 0)
    def _():
        s = pltpu.async_copy(
            rows_v.at[0],
            out_hbm.at[row, pl.ds(32, _CHUNK)],
            ssems[0],
        )
        s.wait()

    @pl.when(part == 1)
    def _():
        s = pltpu.async_copy(
            rows_v.at[0, pl.ds(0, 5)],
            out_hbm.at[row, pl.ds(72, 5)],
            ssems[0],
        )
        s.wait()

    scatters[1].wait()


_mesh = plsc.VectorSubcoreMesh(core_axis_name="c", subcore_axis_name="s")

_emb_lookup = functools.partial(
    pl.kernel,
    mesh=_mesh,
    out_type=jax.ShapeDtypeStruct((16, _T, D_MODEL), jnp.float32),
    scratch_types=[
        pltpu.VMEM((_BLK,), jnp.int32),
        pltpu.VMEM((2, _CHUNK, D_MODEL), jnp.float32),
        pltpu.SemaphoreType.DMA,
        pltpu.SemaphoreType.DMA,
        pltpu.SemaphoreType.DMA,
        pltpu.SemaphoreType.DMA,
    ],
)(_emb_body)


@jax.jit
def kernel(test_input, emb_table):
    idx = test_input.reshape(-1).astype(jnp.int32)
    idx_blocks = jnp.take(idx, jnp.asarray(_PERM))
    return _emb_lookup(idx_blocks, emb_table)


# speedup vs baseline: 1.7799x; 1.0970x over previous
import functools

import jax
import jax.numpy as jnp
from jax import lax
from jax.experimental import pallas as pl
from jax.experimental.pallas import tpu as pltpu
from jax.experimental.pallas import tpu_sc as plsc

VOCAB = 32128
D_MODEL = 4096

_NC = 2
_NS = 16
_NW = _NC * _NS

_T = 77
_CHUNK = 8


def _emb_body(idx_hbm, table_hbm, out_hbm, idx_v, rows_v, g0, g1, s0, s1):
    wid = lax.axis_index("s") * _NC + lax.axis_index("c")
    row = wid // 2
    part = wid % 2
    t0 = part * 40
    base = row * _T + t0
    limit = 47 - part * 11

    lane = lax.iota(jnp.int32, 16)
    stages = []
    for k in range(3):
        pos = base + jnp.minimum(lane + 16 * k, limit)
        stages.append(
            pltpu.async_copy(idx_hbm.at[pos], idx_v.at[pl.ds(16 * k, 16)], g0)
        )
    for st in stages:
        st.wait()

    gsems = (g0, g1)
    ssems = (s0, s1)
    gathers = [None, None]
    scatters = [None, None]
    for c in range(5):
        b = c % 2
        if scatters[b] is not None:
            scatters[b].wait()
        gathers[b] = pltpu.async_copy(
            table_hbm.at[idx_v.at[pl.ds(c * _CHUNK, _CHUNK)]],
            rows_v.at[b],
            gsems[b],
        )
        if c >= 1:
            bp = (c - 1) % 2
            gathers[bp].wait()
            scatters[bp] = pltpu.async_copy(
                rows_v.at[bp],
                out_hbm.at[row, pl.ds(t0 + (c - 1) * _CHUNK, _CHUNK)],
                ssems[bp],
            )
    gathers[0].wait()

    @pl.when(part == 0)
    def _():
        s = pltpu.async_copy(
            rows_v.at[0],
            out_hbm.at[row, pl.ds(32, _CHUNK)],
            ssems[0],
        )
        s.wait()

    @pl.when(part == 1)
    def _():
        s = pltpu.async_copy(
            rows_v.at[0, pl.ds(0, 5)],
            out_hbm.at[row, pl.ds(72, 5)],
            ssems[0],
        )
        s.wait()

    scatters[1].wait()


_mesh = plsc.VectorSubcoreMesh(core_axis_name="c", subcore_axis_name="s")

_emb_lookup = functools.partial(
    pl.kernel,
    mesh=_mesh,
    out_type=jax.ShapeDtypeStruct((16, _T, D_MODEL), jnp.float32),
    scratch_types=[
        pltpu.VMEM((48,), jnp.int32),
        pltpu.VMEM((2, _CHUNK, D_MODEL), jnp.float32),
        pltpu.SemaphoreType.DMA,
        pltpu.SemaphoreType.DMA,
        pltpu.SemaphoreType.DMA,
        pltpu.SemaphoreType.DMA,
    ],
)(_emb_body)


@jax.jit
def kernel(test_input, emb_table):
    idx = test_input.reshape(-1).astype(jnp.int32)
    return _emb_lookup(idx, emb_table)


# speedup vs baseline: 2.8226x; 1.5858x over previous
import functools

import jax
import jax.numpy as jnp
from jax import lax
from jax.experimental import pallas as pl
from jax.experimental.pallas import tpu as pltpu
from jax.experimental.pallas import tpu_sc as plsc

VOCAB = 32128
D_MODEL = 4096

_NC = 2
_NS = 16
_NW = _NC * _NS

_T = 77
_CHUNK = 8


def _emb_body(idx_hbm, table_hbm, out_hbm, idx_v, rows_v, g0, g1, s0, s1):
    wid = lax.axis_index("s") * _NC + lax.axis_index("c")
    row = wid // 2
    part = wid % 2
    t0 = part * 40
    base = row * _T + t0
    limit = 47 - part * 11

    lane = lax.iota(jnp.int32, 16)
    stages = []
    for k in range(3):
        pos = base + jnp.minimum(lane + 16 * k, limit)
        stages.append(
            pltpu.async_copy(idx_hbm.at[pos], idx_v.at[pl.ds(16 * k, 16)], g0)
        )
    for st in stages:
        st.wait()

    gsems = (g0, g1)
    ssems = (s0, s1)
    gathers = [None, None]
    scatters = [None, None]
    for c in range(5):
        b = c % 2
        if scatters[b] is not None:
            scatters[b].wait()
        gathers[b] = pltpu.async_copy(
            table_hbm.at[idx_v.at[pl.ds(c * _CHUNK, _CHUNK)]],
            rows_v.at[b],
            gsems[b],
        )
        if c >= 1:
            bp = (c - 1) % 2
            gathers[bp].wait()
            scatters[bp] = pltpu.async_copy(
                rows_v.at[bp],
                out_hbm.at[pl.ds(t0 + (c - 1) * _CHUNK, _CHUNK), row],
                ssems[bp],
            )
    gathers[0].wait()

    @pl.when(part == 0)
    def _():
        s = pltpu.async_copy(
            rows_v.at[0],
            out_hbm.at[pl.ds(32, _CHUNK), row],
            ssems[0],
        )
        s.wait()

    @pl.when(part == 1)
    def _():
        s = pltpu.async_copy(
            rows_v.at[0, pl.ds(0, 5)],
            out_hbm.at[pl.ds(72, 5), row],
            ssems[0],
        )
        s.wait()

    scatters[1].wait()


_mesh = plsc.VectorSubcoreMesh(core_axis_name="c", subcore_axis_name="s")

_emb_lookup = functools.partial(
    pl.kernel,
    mesh=_mesh,
    out_type=jax.ShapeDtypeStruct((_T, 16, D_MODEL), jnp.float32),
    scratch_types=[
        pltpu.VMEM((48,), jnp.int32),
        pltpu.VMEM((2, _CHUNK, D_MODEL), jnp.float32),
        pltpu.SemaphoreType.DMA,
        pltpu.SemaphoreType.DMA,
        pltpu.SemaphoreType.DMA,
        pltpu.SemaphoreType.DMA,
    ],
)(_emb_body)


@jax.jit
def kernel(test_input, emb_table):
    idx = test_input.reshape(-1).astype(jnp.int32)
    out_tb = _emb_lookup(idx, emb_table)
    return jnp.swapaxes(out_tb, 0, 1)


# speedup vs baseline: 2.8509x; 1.0100x over previous
import functools

import jax
import jax.numpy as jnp
from jax import lax
from jax.experimental import pallas as pl
from jax.experimental.pallas import tpu as pltpu
from jax.experimental.pallas import tpu_sc as plsc

VOCAB = 32128
D_MODEL = 4096

_NC = 2
_NS = 16
_NW = _NC * _NS

_T = 77
_CHUNK = 8


def _emb_body(idx_hbm, table_hbm, out_hbm, idx_v, rows_v, g0, g1, g2, s0, s1, s2):
    wid = lax.axis_index("s") * _NC + lax.axis_index("c")
    row = wid // 2
    part = wid % 2
    t0 = part * 40
    base = row * _T + t0
    limit = 47 - part * 11

    lane = lax.iota(jnp.int32, 16)
    stages = []
    for k in range(3):
        pos = base + jnp.minimum(lane + 16 * k, limit)
        stages.append(
            pltpu.async_copy(idx_hbm.at[pos], idx_v.at[pl.ds(16 * k, 16)], g0)
        )
    for st in stages:
        st.wait()

    gsems = (g0, g1, g2)
    ssems = (s0, s1, s2)
    gathers = [None, None, None]
    scatters = [None, None, None]
    for c in range(5):
        b = c % 3
        if scatters[b] is not None:
            scatters[b].wait()
        gathers[b] = pltpu.async_copy(
            table_hbm.at[idx_v.at[pl.ds(c * _CHUNK, _CHUNK)]],
            rows_v.at[b],
            gsems[b],
        )
        if c >= 1:
            bp = (c - 1) % 3
            gathers[bp].wait()
            scatters[bp] = pltpu.async_copy(
                rows_v.at[bp],
                out_hbm.at[pl.ds(t0 + (c - 1) * _CHUNK, _CHUNK), row],
                ssems[bp],
            )
    gathers[1].wait()

    @pl.when(part == 0)
    def _():
        s = pltpu.async_copy(
            rows_v.at[1],
            out_hbm.at[pl.ds(32, _CHUNK), row],
            ssems[1],
        )
        s.wait()

    @pl.when(part == 1)
    def _():
        s = pltpu.async_copy(
            rows_v.at[1, pl.ds(0, 5)],
            out_hbm.at[pl.ds(72, 5), row],
            ssems[1],
        )
        s.wait()

    scatters[2].wait()
    scatters[0].wait()


_mesh = plsc.VectorSubcoreMesh(core_axis_name="c", subcore_axis_name="s")

_emb_lookup = functools.partial(
    pl.kernel,
    mesh=_mesh,
    out_type=jax.ShapeDtypeStruct((_T, 16, D_MODEL), jnp.float32),
    scratch_types=[
        pltpu.VMEM((48,), jnp.int32),
        pltpu.VMEM((3, _CHUNK, D_MODEL), jnp.float32),
        pltpu.SemaphoreType.DMA,
        pltpu.SemaphoreType.DMA,
        pltpu.SemaphoreType.DMA,
        pltpu.SemaphoreType.DMA,
        pltpu.SemaphoreType.DMA,
        pltpu.SemaphoreType.DMA,
    ],
)(_emb_body)


@jax.jit
def kernel(test_input, emb_table):
    idx = test_input.reshape(-1).astype(jnp.int32)
    out_tb = _emb_lookup(idx, emb_table)
    return jnp.swapaxes(out_tb, 0, 1)
